# baseline (speedup 1.0000x reference)
import functools

import jax
import jax.numpy as jnp
from jax import lax
from jax.experimental import pallas as pl
from jax.experimental.pallas import tpu as pltpu
from jax.experimental.pallas import tpu_sc as plsc

N = 10000
E = 320000
D = 128
H = 128
C = 40
P = 1024

NP = 10240
NC = 2
NS = 16
NW = NC * NS
CHUNK = 128
EPT_CHUNKS = 80
EPT = EPT_CHUNKS * CHUNK
E_PAD = EPT * NW
ROWS_PER_SUB = NP // NS
P_PER_SUB = P // NS
EPS = 1e-5


def _sc_mesh():
  return plsc.VectorSubcoreMesh(core_axis_name="c", subcore_axis_name="s")


def _unpack(ed, j, sidx, didx):
  row = ed.at[j]
  for k in range(CHUNK // 16):
    v = row[pl.ds(16 * k, 16)]
    sidx[pl.ds(16 * k, 16)] = lax.bitwise_and(v, 0xFFFF)
    didx[pl.ds(16 * k, 16)] = lax.shift_right_logical(v, 16)


def _agg_body(prime, h_ref, ed_ref, z_ref, mask_ref, out_ref,
              ed, sidx_a, didx_a, sidx_b, didx_b, rows_a, rows_b,
              acc, sem_a, sem_b):
  c = lax.axis_index("c")
  s = lax.axis_index("s")
  wid = c * NS + s

  pltpu.sync_copy(z_ref, acc.at[pl.ds(s * ROWS_PER_SUB, ROWS_PER_SUB)])

  pltpu.sync_copy(ed_ref.at[pl.ds(wid * EPT_CHUNKS, EPT_CHUNKS)], ed)
  plsc.subcore_barrier()

  def gather(sidx, rows, gsem):
    pltpu.async_copy(h_ref.at[sidx], rows, gsem)

  def gwait(sidx, rows, gsem):
    pltpu.make_async_copy(h_ref.at[sidx], rows, gsem).wait()

  def scat(rows, didx, ssem):
    pltpu.async_copy(rows, acc.at[didx], ssem, add=True)

  def swait(rows, didx, ssem):
    pltpu.make_async_copy(rows, acc.at[didx], ssem).wait()

  _unpack(ed, 0, sidx_a, didx_a)
  gather(sidx_a, rows_a, sem_a)
  _unpack(ed, 1, sidx_b, didx_b)
  gather(sidx_b, rows_b, sem_b)
  gwait(sidx_a, rows_a, sem_a)
  scat(rows_a, didx_a, sem_a)
  gwait(sidx_b, rows_b, sem_b)
  scat(rows_b, didx_b, sem_b)

  def pair(j, carry):
    c0 = 2 * j
    c1 = c0 + 1
    swait(rows_a, didx_a, sem_a)
    _unpack(ed, c0, sidx_a, didx_a)
    gather(sidx_a, rows_a, sem_a)
    swait(rows_b, didx_b, sem_b)
    _unpack(ed, c1, sidx_b, didx_b)
    gather(sidx_b, rows_b, sem_b)
    gwait(sidx_a, rows_a, sem_a)
    scat(rows_a, didx_a, sem_a)
    gwait(sidx_b, rows_b, sem_b)
    scat(rows_b, didx_b, sem_b)
    return carry

  lax.fori_loop(1, EPT_CHUNKS // 2, pair, 0)
  swait(rows_a, didx_a, sem_a)
  swait(rows_b, didx_b, sem_b)
  plsc.subcore_barrier()

  if prime:
    midx = sidx_a.at[pl.ds(0, P_PER_SUB)]
    mrows = rows_a.at[pl.ds(0, P_PER_SUB)]
    pltpu.sync_copy(mask_ref.at[pl.ds(s * P_PER_SUB, P_PER_SUB)], midx)
    pltpu.async_copy(acc.at[midx], mrows, sem_a).wait()
    pltpu.sync_copy(mrows, out_ref.at[pl.ds(c * P + s * P_PER_SUB, P_PER_SUB)])
  else:
    pltpu.sync_copy(acc.at[pl.ds(s * ROWS_PER_SUB, ROWS_PER_SUB)],
                    out_ref.at[pl.ds(c * NP + s * ROWS_PER_SUB, ROWS_PER_SUB)])


def _make_agg(prime):
  out_rows = 2 * P if prime else 2 * NP
  body = functools.partial(_agg_body, prime)
  return pl.kernel(
      body,
      out_type=jax.ShapeDtypeStruct((out_rows, H), jnp.float32),
      mesh=_sc_mesh(),
      scratch_types=[
          pltpu.VMEM((EPT_CHUNKS, CHUNK), jnp.int32),
          pltpu.VMEM((CHUNK,), jnp.int32),
          pltpu.VMEM((CHUNK,), jnp.int32),
          pltpu.VMEM((CHUNK,), jnp.int32),
          pltpu.VMEM((CHUNK,), jnp.int32),
          pltpu.VMEM((CHUNK, H), jnp.float32),
          pltpu.VMEM((CHUNK, H), jnp.float32),
          pltpu.VMEM_SHARED((NP, H), jnp.float32),
          pltpu.SemaphoreType.DMA,
          pltpu.SemaphoreType.DMA,
      ],
      name="sc_edge_agg_prime" if prime else "sc_edge_agg",
  )


def _mm_body(x_ref, w_ref, o_ref):
  o_ref[...] = jnp.dot(x_ref[...], w_ref[...],
                       preferred_element_type=jnp.float32)


def _tc_matmul(x, w):
  m = x.shape[0]
  bm = 1280
  grid = m // bm
  return pl.pallas_call(
      _mm_body,
      grid=(grid,),
      in_specs=[
          pl.BlockSpec((bm, D), lambda i: (i, 0)),
          pl.BlockSpec((D, H), lambda i: (0, 0)),
      ],
      out_specs=pl.BlockSpec((bm, H), lambda i: (i, 0)),
      out_shape=jax.ShapeDtypeStruct((m, H), jnp.float32),
  )(x, w)


def _ln_relu(a, g, b):
  mu = jnp.mean(a, axis=-1, keepdims=True)
  var = jnp.mean((a - mu) ** 2, axis=-1, keepdims=True)
  hn = (a - mu) * lax.rsqrt(var + EPS) * g + b
  return jnp.maximum(hn, 0.0)


def _mid_body(p0_ref, p1_ref, g_ref, b_ref, w_ref, o_ref):
  a = p0_ref[...] + p1_ref[...]
  h = _ln_relu(a, g_ref[...], b_ref[...])
  o_ref[...] = jnp.dot(h, w_ref[...], preferred_element_type=jnp.float32)


def _tc_mid(p0, p1, g, b, w):
  m = p0.shape[0]
  bm = 1280
  grid = m // bm
  return pl.pallas_call(
      _mid_body,
      grid=(grid,),
      in_specs=[
          pl.BlockSpec((bm, H), lambda i: (i, 0)),
          pl.BlockSpec((bm, H), lambda i: (i, 0)),
          pl.BlockSpec((1, H), lambda i: (0, 0)),
          pl.BlockSpec((1, H), lambda i: (0, 0)),
          pl.BlockSpec((H, H), lambda i: (0, 0)),
      ],
      out_specs=pl.BlockSpec((bm, H), lambda i: (i, 0)),
      out_shape=jax.ShapeDtypeStruct((m, H), jnp.float32),
  )(p0, p1, g, b, w)


def _final_body(p0_ref, p1_ref, g_ref, b_ref, w_ref, bl_ref, o_ref):
  a = p0_ref[...] + p1_ref[...]
  h = _ln_relu(a, g_ref[...], b_ref[...])
  z = jnp.dot(h, w_ref[...], preferred_element_type=jnp.float32) + bl_ref[...]
  col = lax.broadcasted_iota(jnp.int32, z.shape, 1)
  z = jnp.where(col < C, z, -1e30)
  m = jnp.max(z, axis=-1, keepdims=True)
  lse = jnp.log(jnp.sum(jnp.exp(z - m), axis=-1, keepdims=True))
  o_ref[...] = z - m - lse


def _tc_final(p0, p1, g, b, w, bl):
  return pl.pallas_call(
      _final_body,
      out_shape=jax.ShapeDtypeStruct((P, H), jnp.float32),
  )(p0, p1, g, b, w, bl)


def _dummy_body(a_ref, o_ref, buf, sem):
  s = lax.axis_index("s")
  @pl.when(jnp.logical_and(s == 0, lax.axis_index("c") == 0))
  def _():
    pltpu.sync_copy(a_ref, buf)
    pltpu.sync_copy(buf, o_ref)


_dummy = pl.kernel(
    _dummy_body,
    out_type=jax.ShapeDtypeStruct((8, H), jnp.float32),
    mesh=_sc_mesh(),
    scratch_types=[
        pltpu.VMEM((8, H), jnp.float32),
        pltpu.SemaphoreType.DMA,
    ],
    name="sc_dummy",
)


def kernel(x, edge_index, output_node_mask, W0, ln0_g, ln0_b, W1, ln1_g,
           ln1_b, Wl, bl):
  xp = jnp.pad(x, ((0, NP - N), (0, 0)))
  dst = jnp.concatenate(
      [edge_index[0], jnp.full((E_PAD - E,), NP - 1, jnp.int32)])
  src = jnp.concatenate([edge_index[1], jnp.zeros((E_PAD - E,), jnp.int32)])
  packed = jnp.bitwise_or(src, jnp.left_shift(dst, 16))
  ed2d = packed.reshape(E_PAD // CHUNK, CHUNK)
  zeros = jnp.zeros((ROWS_PER_SUB, H), jnp.float32)
  dummy_mask = jnp.zeros((P,), jnp.int32)

  g0 = ln0_g.reshape(1, H)
  b0 = ln0_b.reshape(1, H)
  g1 = ln1_g.reshape(1, H)
  b1 = ln1_b.reshape(1, H)
  Wl_pad = jnp.pad(Wl, ((0, 0), (0, H - C)))
  bl_pad = jnp.pad(bl, ((0, H - C))).reshape(1, H)

  h = _tc_matmul(xp, W0)
  h = h + 0.0 * jnp.pad(_dummy(h[:8]), ((0, NP - 8), (0, 0)))
  parts = _make_agg(False)(h, ed2d, zeros, dummy_mask)
  h2 = _tc_mid(parts[:NP], parts[NP:], g0, b0, W1)
  parts2 = _make_agg(True)(h2, ed2d, zeros, output_node_mask)
  outp = _tc_final(parts2[:P], parts2[P:], g1, b1, Wl_pad, bl_pad)
  return outp[:, :C]


# speedup vs baseline: 1.0287x; 1.0287x over previous
import functools

import jax
import jax.numpy as jnp
from jax import lax
from jax.experimental import pallas as pl
from jax.experimental.pallas import tpu as pltpu
from jax.experimental.pallas import tpu_sc as plsc

N = 10000
E = 320000
D = 128
H = 128
C = 40
P = 1024

NP = 10240
NC = 2
NS = 16
NW = NC * NS
CHUNK = 64
NBUF = 4
EPR = 128
EPT_ROWS = 80
EPT = EPT_ROWS * EPR
NCH = EPT // CHUNK
E_PAD = EPT * NW
ROWS_PER_SUB = NP // NS
P_PER_SUB = P // NS
EPS = 1e-5


def _sc_mesh():
  return plsc.VectorSubcoreMesh(core_axis_name="c", subcore_axis_name="s")


def _unpack(ed, row, half, sidx, didx):
  r = ed.at[row]
  for k in range(CHUNK // 16):
    v = r[pl.ds(64 * half + 16 * k, 16)]
    sidx[pl.ds(16 * k, 16)] = lax.bitwise_and(v, 0xFFFF)
    didx[pl.ds(16 * k, 16)] = lax.shift_right_logical(v, 16)


def _agg_body(prime, h_ref, ed_ref, z_ref, mask_ref, out_ref,
              ed, sidx, didx, rows, acc, sems):
  c = lax.axis_index("c")
  s = lax.axis_index("s")
  wid = c * NS + s

  pltpu.sync_copy(z_ref, acc.at[pl.ds(s * ROWS_PER_SUB, ROWS_PER_SUB)])

  pltpu.sync_copy(ed_ref.at[pl.ds(wid * EPT_ROWS, EPT_ROWS)], ed)
  plsc.subcore_barrier()

  def unpack_c(cr, ch, q):
    _unpack(ed, cr, ch, sidx[q], didx[q])

  def gather(q):
    pltpu.async_copy(h_ref.at[sidx[q]], rows[q], sems[q])

  def gwait(q):
    pltpu.make_async_copy(h_ref.at[sidx[q]], rows[q], sems[q]).wait()

  def scat(q):
    pltpu.async_copy(rows[q], acc.at[didx[q]], sems[q], add=True)

  def swait(q):
    pltpu.make_async_copy(rows[q], acc.at[didx[q]], sems[q]).wait()

  for q in range(NBUF):
    unpack_c(q // 2, q % 2, q)
    gather(q)

  def body(j, carry):
    for q in range(NBUF):
      gwait(q)
      scat(q)
    for q in range(NBUF):
      swait(q)
      unpack_c(2 * j + q // 2, q % 2, q)
      gather(q)
    return carry

  lax.fori_loop(1, NCH // NBUF, body, 0)
  for q in range(NBUF):
    gwait(q)
    scat(q)
  for q in range(NBUF):
    swait(q)
  plsc.subcore_barrier()

  if prime:
    midx = sidx[0]
    mrows = rows[0]
    pltpu.sync_copy(mask_ref.at[pl.ds(s * P_PER_SUB, P_PER_SUB)], midx)
    pltpu.async_copy(acc.at[midx], mrows, sems[0]).wait()
    pltpu.sync_copy(mrows, out_ref.at[pl.ds(c * P + s * P_PER_SUB, P_PER_SUB)])
  else:
    pltpu.sync_copy(acc.at[pl.ds(s * ROWS_PER_SUB, ROWS_PER_SUB)],
                    out_ref.at[pl.ds(c * NP + s * ROWS_PER_SUB, ROWS_PER_SUB)])


def _make_agg(prime):
  out_rows = 2 * P if prime else 2 * NP
  body = functools.partial(_agg_body, prime)
  return pl.kernel(
      body,
      out_type=jax.ShapeDtypeStruct((out_rows, H), jnp.float32),
      mesh=_sc_mesh(),
      scratch_types=[
          pltpu.VMEM((EPT_ROWS, EPR), jnp.int32),
          [pltpu.VMEM((CHUNK,), jnp.int32) for _ in range(NBUF)],
          [pltpu.VMEM((CHUNK,), jnp.int32) for _ in range(NBUF)],
          [pltpu.VMEM((CHUNK, H), jnp.float32) for _ in range(NBUF)],
          pltpu.VMEM_SHARED((NP, H), jnp.float32),
          [pltpu.SemaphoreType.DMA for _ in range(NBUF)],
      ],
      name="sc_edge_agg_prime" if prime else "sc_edge_agg",
  )


def _mm_body(x_ref, w_ref, o_ref):
  o_ref[...] = jnp.dot(x_ref[...], w_ref[...],
                       preferred_element_type=jnp.float32)


def _tc_matmul(x, w):
  m = x.shape[0]
  bm = 1280
  grid = m // bm
  return pl.pallas_call(
      _mm_body,
      grid=(grid,),
      in_specs=[
          pl.BlockSpec((bm, D), lambda i: (i, 0)),
          pl.BlockSpec((D, H), lambda i: (0, 0)),
      ],
      out_specs=pl.BlockSpec((bm, H), lambda i: (i, 0)),
      out_shape=jax.ShapeDtypeStruct((m, H), jnp.float32),
  )(x, w)


def _ln_relu(a, g, b):
  mu = jnp.mean(a, axis=-1, keepdims=True)
  var = jnp.mean((a - mu) ** 2, axis=-1, keepdims=True)
  hn = (a - mu) * lax.rsqrt(var + EPS) * g + b
  return jnp.maximum(hn, 0.0)


def _mid_body(p0_ref, p1_ref, g_ref, b_ref, w_ref, o_ref):
  a = p0_ref[...] + p1_ref[...]
  h = _ln_relu(a, g_ref[...], b_ref[...])
  o_ref[...] = jnp.dot(h, w_ref[...], preferred_element_type=jnp.float32)


def _tc_mid(p0, p1, g, b, w):
  m = p0.shape[0]
  bm = 1280
  grid = m // bm
  return pl.pallas_call(
      _mid_body,
      grid=(grid,),
      in_specs=[
          pl.BlockSpec((bm, H), lambda i: (i, 0)),
          pl.BlockSpec((bm, H), lambda i: (i, 0)),
          pl.BlockSpec((1, H), lambda i: (0, 0)),
          pl.BlockSpec((1, H), lambda i: (0, 0)),
          pl.BlockSpec((H, H), lambda i: (0, 0)),
      ],
      out_specs=pl.BlockSpec((bm, H), lambda i: (i, 0)),
      out_shape=jax.ShapeDtypeStruct((m, H), jnp.float32),
  )(p0, p1, g, b, w)


def _final_body(p0_ref, p1_ref, g_ref, b_ref, w_ref, bl_ref, o_ref):
  a = p0_ref[...] + p1_ref[...]
  h = _ln_relu(a, g_ref[...], b_ref[...])
  z = jnp.dot(h, w_ref[...], preferred_element_type=jnp.float32) + bl_ref[...]
  col = lax.broadcasted_iota(jnp.int32, z.shape, 1)
  z = jnp.where(col < C, z, -1e30)
  m = jnp.max(z, axis=-1, keepdims=True)
  lse = jnp.log(jnp.sum(jnp.exp(z - m), axis=-1, keepdims=True))
  o_ref[...] = z - m - lse


def _tc_final(p0, p1, g, b, w, bl):
  return pl.pallas_call(
      _final_body,
      out_shape=jax.ShapeDtypeStruct((P, H), jnp.float32),
  )(p0, p1, g, b, w, bl)


def kernel(x, edge_index, output_node_mask, W0, ln0_g, ln0_b, W1, ln1_g,
           ln1_b, Wl, bl):
  xp = jnp.pad(x, ((0, NP - N), (0, 0)))
  dst = jnp.concatenate(
      [edge_index[0], jnp.full((E_PAD - E,), NP - 1, jnp.int32)])
  src = jnp.concatenate([edge_index[1], jnp.zeros((E_PAD - E,), jnp.int32)])
  packed = jnp.bitwise_or(src, jnp.left_shift(dst, 16))
  ed2d = packed.reshape(E_PAD // EPR, EPR)
  zeros = jnp.zeros((ROWS_PER_SUB, H), jnp.float32)
  dummy_mask = jnp.zeros((P,), jnp.int32)

  g0 = ln0_g.reshape(1, H)
  b0 = ln0_b.reshape(1, H)
  g1 = ln1_g.reshape(1, H)
  b1 = ln1_b.reshape(1, H)
  Wl_pad = jnp.pad(Wl, ((0, 0), (0, H - C)))
  bl_pad = jnp.pad(bl, ((0, H - C))).reshape(1, H)

  h = _tc_matmul(xp, W0)
  parts = _make_agg(False)(h, ed2d, zeros, dummy_mask)
  h2 = _tc_mid(parts[:NP], parts[NP:], g0, b0, W1)
  parts2 = _make_agg(True)(h2, ed2d, zeros, output_node_mask)
  outp = _tc_final(parts2[:P], parts2[P:], g1, b1, Wl_pad, bl_pad)
  return outp[:, :C]


# speedup vs baseline: 1.5407x; 1.4977x over previous
import functools

import jax
import jax.numpy as jnp
from jax import lax
from jax.experimental import pallas as pl
from jax.experimental.pallas import tpu as pltpu
from jax.experimental.pallas import tpu_sc as plsc

N = 10000
E = 320000
D = 128
H = 128
C = 40
P = 1024

NP = 10240
NC = 2
NS = 16
NW = NC * NS
CHUNK = 64
NBUF = 4
EPR = 128
EPT_ROWS = 80
EPT = EPT_ROWS * EPR
NCH = EPT // CHUNK
E_PAD = EPT * NW
ROWS_PER_SUB = NP // NS
P_PER_SUB = P // NS
EPS = 1e-5


def _sc_mesh():
  return plsc.VectorSubcoreMesh(core_axis_name="c", subcore_axis_name="s")


def _unpack(ed, row, half, sidx, didx):
  r = ed.at[row]
  for k in range(CHUNK // 16):
    v = r[pl.ds(64 * half + 16 * k, 16)]
    sidx[pl.ds(16 * k, 16)] = lax.bitwise_and(v, 0xFFFF)
    didx[pl.ds(16 * k, 16)] = lax.shift_right_logical(v, 16)


PR = 1152
PR_PER_SUB = PR // NS


def _prime_body(h_ref, ed_ref, z_ref, nmap_ref, rep_ref, out_ref,
                ed, sidx_a, didx_a, ldst_a, sidx_b, didx_b, ldst_b,
                rows_a, rows_b, nmap, table, acc,
                sem_a, sem_b, lsem_a, lsem_b):
  c = lax.axis_index("c")
  s = lax.axis_index("s")
  wid = c * NS + s
  rs = s * ROWS_PER_SUB

  pltpu.sync_copy(h_ref.at[pl.ds(rs, ROWS_PER_SUB)],
                  table.at[pl.ds(rs, ROWS_PER_SUB)])
  pltpu.sync_copy(z_ref, acc.at[pl.ds(s * PR_PER_SUB, PR_PER_SUB)])
  pltpu.sync_copy(nmap_ref.at[pl.ds(rs, ROWS_PER_SUB)],
                  nmap.at[pl.ds(rs, ROWS_PER_SUB)])
  plsc.subcore_barrier()

  def lookup(didx, ldst, lsem):
    pltpu.async_copy(nmap.at[didx], ldst, lsem)

  def lwait(didx, ldst, lsem):
    pltpu.make_async_copy(nmap.at[didx], ldst, lsem).wait()

  def gather(sidx, rows, gsem):
    pltpu.async_copy(table.at[sidx], rows, gsem)

  def gwait(sidx, rows, gsem):
    pltpu.make_async_copy(table.at[sidx], rows, gsem).wait()

  def scat(rows, ldst, ssem):
    pltpu.async_copy(rows, acc.at[ldst], ssem, add=True)

  def swait(rows, ldst, ssem):
    pltpu.make_async_copy(rows, acc.at[ldst], ssem).wait()

  def prep(cr, ch, sidx, didx, ldst, lsem):
    _unpack(ed, cr, ch, sidx, didx)
    lookup(didx, ldst, lsem)

  def pair(j, carry):
    swait(rows_a, ldst_a, sem_a)
    prep(j, 0, sidx_a, didx_a, ldst_a, lsem_a)
    gather(sidx_a, rows_a, sem_a)
    swait(rows_b, ldst_b, sem_b)
    prep(j, 1, sidx_b, didx_b, ldst_b, lsem_b)
    gather(sidx_b, rows_b, sem_b)
    gwait(sidx_a, rows_a, sem_a)
    lwait(didx_a, ldst_a, lsem_a)
    scat(rows_a, ldst_a, sem_a)
    gwait(sidx_b, rows_b, sem_b)
    lwait(didx_b, ldst_b, lsem_b)
    scat(rows_b, ldst_b, sem_b)
    return carry

  for p in range(2):
    pltpu.sync_copy(
        ed_ref.at[pl.ds(wid * EPT_ROWS + p * (EPT_ROWS // 2), EPT_ROWS // 2)],
        ed)
    prep(0, 0, sidx_a, didx_a, ldst_a, lsem_a)
    gather(sidx_a, rows_a, sem_a)
    prep(0, 1, sidx_b, didx_b, ldst_b, lsem_b)
    gather(sidx_b, rows_b, sem_b)
    gwait(sidx_a, rows_a, sem_a)
    lwait(didx_a, ldst_a, lsem_a)
    scat(rows_a, ldst_a, sem_a)
    gwait(sidx_b, rows_b, sem_b)
    lwait(didx_b, ldst_b, lsem_b)
    scat(rows_b, ldst_b, sem_b)
    lax.fori_loop(1, EPT_ROWS // 2, pair, 0)
    swait(rows_a, ldst_a, sem_a)
    swait(rows_b, ldst_b, sem_b)
  plsc.subcore_barrier()

  midx = sidx_a.at[pl.ds(0, P_PER_SUB)]
  mrows = rows_a.at[pl.ds(0, P_PER_SUB)]
  pltpu.sync_copy(rep_ref.at[pl.ds(s * P_PER_SUB, P_PER_SUB)], midx)
  pltpu.async_copy(acc.at[midx], mrows, sem_a).wait()
  pltpu.sync_copy(mrows, out_ref.at[pl.ds(c * P + s * P_PER_SUB, P_PER_SUB)])


def _make_prime():
  return pl.kernel(
      _prime_body,
      out_type=jax.ShapeDtypeStruct((2 * P, H), jnp.float32),
      mesh=_sc_mesh(),
      scratch_types=[
          pltpu.VMEM((EPT_ROWS // 2, EPR), jnp.int32),
          pltpu.VMEM((CHUNK,), jnp.int32),
          pltpu.VMEM((CHUNK,), jnp.int32),
          pltpu.VMEM((CHUNK,), jnp.int32),
          pltpu.VMEM((CHUNK,), jnp.int32),
          pltpu.VMEM((CHUNK,), jnp.int32),
          pltpu.VMEM((CHUNK,), jnp.int32),
          pltpu.VMEM((CHUNK, H), jnp.float32),
          pltpu.VMEM((CHUNK, H), jnp.float32),
          pltpu.VMEM_SHARED((NP,), jnp.int32),
          pltpu.VMEM_SHARED((NP, H), jnp.float32),
          pltpu.VMEM_SHARED((PR, H), jnp.float32),
          pltpu.SemaphoreType.DMA,
          pltpu.SemaphoreType.DMA,
          pltpu.SemaphoreType.DMA,
          pltpu.SemaphoreType.DMA,
      ],
      name="sc_edge_agg_prime",
  )


def _agg_body(prime, h_ref, ed_ref, z_ref, mask_ref, out_ref,
              ed, sidx, didx, rows, acc, sems):
  c = lax.axis_index("c")
  s = lax.axis_index("s")
  wid = c * NS + s

  pltpu.sync_copy(z_ref, acc.at[pl.ds(s * ROWS_PER_SUB, ROWS_PER_SUB)])

  pltpu.sync_copy(ed_ref.at[pl.ds(wid * EPT_ROWS, EPT_ROWS)], ed)
  plsc.subcore_barrier()

  def unpack_c(cr, ch, q):
    _unpack(ed, cr, ch, sidx[q], didx[q])

  def gather(q):
    pltpu.async_copy(h_ref.at[sidx[q]], rows[q], sems[q])

  def gwait(q):
    pltpu.make_async_copy(h_ref.at[sidx[q]], rows[q], sems[q]).wait()

  def scat(q):
    pltpu.async_copy(rows[q], acc.at[didx[q]], sems[q], add=True)

  def swait(q):
    pltpu.make_async_copy(rows[q], acc.at[didx[q]], sems[q]).wait()

  for q in range(NBUF):
    unpack_c(q // 2, q % 2, q)
    gather(q)

  def body(j, carry):
    for q in range(NBUF):
      gwait(q)
      scat(q)
    for q in range(NBUF):
      swait(q)
      unpack_c(2 * j + q // 2, q % 2, q)
      gather(q)
    return carry

  lax.fori_loop(1, NCH // NBUF, body, 0)
  for q in range(NBUF):
    gwait(q)
    scat(q)
  for q in range(NBUF):
    swait(q)
  plsc.subcore_barrier()

  if prime:
    midx = sidx[0]
    mrows = rows[0]
    pltpu.sync_copy(mask_ref.at[pl.ds(s * P_PER_SUB, P_PER_SUB)], midx)
    pltpu.async_copy(acc.at[midx], mrows, sems[0]).wait()
    pltpu.sync_copy(mrows, out_ref.at[pl.ds(c * P + s * P_PER_SUB, P_PER_SUB)])
  else:
    pltpu.sync_copy(acc.at[pl.ds(s * ROWS_PER_SUB, ROWS_PER_SUB)],
                    out_ref.at[pl.ds(c * NP + s * ROWS_PER_SUB, ROWS_PER_SUB)])


def _make_agg(prime):
  out_rows = 2 * P if prime else 2 * NP
  body = functools.partial(_agg_body, prime)
  return pl.kernel(
      body,
      out_type=jax.ShapeDtypeStruct((out_rows, H), jnp.float32),
      mesh=_sc_mesh(),
      scratch_types=[
          pltpu.VMEM((EPT_ROWS, EPR), jnp.int32),
          [pltpu.VMEM((CHUNK,), jnp.int32) for _ in range(NBUF)],
          [pltpu.VMEM((CHUNK,), jnp.int32) for _ in range(NBUF)],
          [pltpu.VMEM((CHUNK, H), jnp.float32) for _ in range(NBUF)],
          pltpu.VMEM_SHARED((NP, H), jnp.float32),
          [pltpu.SemaphoreType.DMA for _ in range(NBUF)],
      ],
      name="sc_edge_agg_prime" if prime else "sc_edge_agg",
  )


def _mm_body(x_ref, w_ref, o_ref):
  o_ref[...] = jnp.dot(x_ref[...], w_ref[...],
                       preferred_element_type=jnp.float32)


def _tc_matmul(x, w):
  m = x.shape[0]
  bm = 1280
  grid = m // bm
  return pl.pallas_call(
      _mm_body,
      grid=(grid,),
      in_specs=[
          pl.BlockSpec((bm, D), lambda i: (i, 0)),
          pl.BlockSpec((D, H), lambda i: (0, 0)),
      ],
      out_specs=pl.BlockSpec((bm, H), lambda i: (i, 0)),
      out_shape=jax.ShapeDtypeStruct((m, H), jnp.float32),
  )(x, w)


def _ln_relu(a, g, b):
  mu = jnp.mean(a, axis=-1, keepdims=True)
  var = jnp.mean((a - mu) ** 2, axis=-1, keepdims=True)
  hn = (a - mu) * lax.rsqrt(var + EPS) * g + b
  return jnp.maximum(hn, 0.0)


def _mid_body(p0_ref, p1_ref, g_ref, b_ref, w_ref, o_ref):
  a = p0_ref[...] + p1_ref[...]
  h = _ln_relu(a, g_ref[...], b_ref[...])
  o_ref[...] = jnp.dot(h, w_ref[...], preferred_element_type=jnp.float32)


def _tc_mid(p0, p1, g, b, w):
  m = p0.shape[0]
  bm = 1280
  grid = m // bm
  return pl.pallas_call(
      _mid_body,
      grid=(grid,),
      in_specs=[
          pl.BlockSpec((bm, H), lambda i: (i, 0)),
          pl.BlockSpec((bm, H), lambda i: (i, 0)),
          pl.BlockSpec((1, H), lambda i: (0, 0)),
          pl.BlockSpec((1, H), lambda i: (0, 0)),
          pl.BlockSpec((H, H), lambda i: (0, 0)),
      ],
      out_specs=pl.BlockSpec((bm, H), lambda i: (i, 0)),
      out_shape=jax.ShapeDtypeStruct((m, H), jnp.float32),
  )(p0, p1, g, b, w)


def _final_body(p0_ref, p1_ref, g_ref, b_ref, w_ref, bl_ref, o_ref):
  a = p0_ref[...] + p1_ref[...]
  h = _ln_relu(a, g_ref[...], b_ref[...])
  z = jnp.dot(h, w_ref[...], preferred_element_type=jnp.float32) + bl_ref[...]
  col = lax.broadcasted_iota(jnp.int32, z.shape, 1)
  z = jnp.where(col < C, z, -1e30)
  m = jnp.max(z, axis=-1, keepdims=True)
  lse = jnp.log(jnp.sum(jnp.exp(z - m), axis=-1, keepdims=True))
  o_ref[...] = z - m - lse


def _tc_final(p0, p1, g, b, w, bl):
  return pl.pallas_call(
      _final_body,
      out_shape=jax.ShapeDtypeStruct((P, H), jnp.float32),
  )(p0, p1, g, b, w, bl)


def kernel(x, edge_index, output_node_mask, W0, ln0_g, ln0_b, W1, ln1_g,
           ln1_b, Wl, bl):
  xp = jnp.pad(x, ((0, NP - N), (0, 0)))
  dst = jnp.concatenate(
      [edge_index[0], jnp.full((E_PAD - E,), NP - 1, jnp.int32)])
  src = jnp.concatenate([edge_index[1], jnp.zeros((E_PAD - E,), jnp.int32)])
  packed = jnp.bitwise_or(src, jnp.left_shift(dst, 16))
  ed2d = packed.reshape(E_PAD // EPR, EPR)
  zeros = jnp.zeros((ROWS_PER_SUB, H), jnp.float32)
  zeros_p = jnp.zeros((PR_PER_SUB, H), jnp.float32)
  dummy_mask = jnp.zeros((P,), jnp.int32)

  g0 = ln0_g.reshape(1, H)
  b0 = ln0_b.reshape(1, H)
  g1 = ln1_g.reshape(1, H)
  b1 = ln1_b.reshape(1, H)
  Wl_pad = jnp.pad(Wl, ((0, 0), (0, H - C)))
  bl_pad = jnp.pad(bl, ((0, H - C))).reshape(1, H)

  h = _tc_matmul(xp, W0)
  parts = _make_agg(False)(h, ed2d, zeros, dummy_mask)
  h2 = _tc_mid(parts[:NP], parts[NP:], g0, b0, W1)
  nmap_arr = jnp.full((NP,), 1024, jnp.int32).at[output_node_mask].set(
      jnp.arange(P, dtype=jnp.int32))
  rep_arr = nmap_arr[output_node_mask]
  parts2 = _make_prime()(h2, ed2d, zeros_p, nmap_arr, rep_arr)
  outp = _tc_final(parts2[:P], parts2[P:], g1, b1, Wl_pad, bl_pad)
  return outp[:, :C]
